# baseline (speedup 1.0000x reference)
import jax
import jax.numpy as jnp
from jax import lax
from jax.experimental import pallas as pl
from jax.experimental.pallas import tpu as pltpu
from jax.experimental.pallas import tpu_sc as plsc

B = 16384
F = 26
V = 100000
VP = 100096

NC = 2
NS = 16
NW = NC * NS
RPW = B // NW
WORDS = RPW * F
QROWS = RPW // 128
NROW = F * QROWS
L = 16
XROWS = B // 128


def _body(xt_hbm, emb_hbm, out_hbm, xcol, valv, outv, sem):
    c = lax.axis_index("c")
    s = lax.axis_index("s")
    wid = s * NC + c

    def xfire(f, _):
        pltpu.async_copy(
            xt_hbm.at[pl.ds(f * XROWS + wid * QROWS, QROWS)],
            xcol.at[pl.ds(f * QROWS, QROWS)],
            sem,
        )
        return 0

    lax.fori_loop(0, F, xfire, 0)
    pltpu.make_async_copy(xt_hbm.at[pl.ds(0, NROW)], xcol, sem).wait()

    def off(j, _):
        base = (j // QROWS) * VP
        for k in range(128 // L):
            xcol[j, pl.ds(k * L, L)] = xcol[j, pl.ds(k * L, L)] + base
        return 0

    lax.fori_loop(0, NROW, off, 0)

    def fire(j, _):
        pltpu.async_copy(emb_hbm.at[xcol.at[j]], valv.at[pl.ds(j * 128, 128)], sem)
        return 0

    lax.fori_loop(0, NROW, fire, 0)
    pltpu.make_async_copy(emb_hbm.at[pl.ds(0, WORDS)], valv, sem).wait()

    def acc(cth, _):
        def add_f(f, a):
            return a + valv[pl.ds(f * RPW + cth * L, L)]

        a = lax.fori_loop(0, F, add_f, jnp.zeros((L,), jnp.float32))
        outv[pl.ds(cth * L, L)] = a
        return 0

    lax.fori_loop(0, RPW // L, acc, 0)
    pltpu.sync_copy(outv, out_hbm.at[pl.ds(wid * RPW, RPW)])


def kernel(X, emb):
    xt = X.T.astype(jnp.int32).reshape(F * XROWS, 128)
    embp = jnp.pad(emb, ((0, 0), (0, VP - V), (0, 0))).reshape(-1)
    mesh = plsc.VectorSubcoreMesh(
        core_axis_name="c", subcore_axis_name="s", num_cores=NC, num_subcores=NS
    )
    out = pl.kernel(
        _body,
        out_type=jax.ShapeDtypeStruct((B,), jnp.float32),
        mesh=mesh,
        scratch_types=[
            pltpu.VMEM((NROW, 128), jnp.int32),
            pltpu.VMEM((WORDS,), jnp.float32),
            pltpu.VMEM((RPW,), jnp.float32),
            pltpu.SemaphoreType.DMA,
        ],
        compiler_params=pltpu.CompilerParams(needs_layout_passes=False),
    )(xt, embp)
    return out.reshape(B, 1)


# speedup vs baseline: 2.1266x; 2.1266x over previous
import jax
import jax.numpy as jnp
from jax import lax
from jax.experimental import pallas as pl
from jax.experimental.pallas import tpu as pltpu
from jax.experimental.pallas import tpu_sc as plsc

B = 16384
F = 26
V = 100000
VP = 100096

NC = 2
NS = 16
NW = NC * NS
RPW = B // NW
WORDS = RPW * F
QROWS = RPW // 128
NROW = F * QROWS
L = 16
XROWS = B // 128


def _body(xt_hbm, emb_hbm, out_hbm, xcol, valv, outv, sem, fsem):
    c = lax.axis_index("c")
    s = lax.axis_index("s")
    wid = s * NC + c

    def xfire(f, _):
        pltpu.async_copy(
            xt_hbm.at[pl.ds(f * XROWS + wid * QROWS, QROWS)],
            xcol.at[pl.ds(f * QROWS, QROWS)],
            sem,
        )
        return 0

    lax.fori_loop(0, F, xfire, 0)
    pltpu.make_async_copy(xt_hbm.at[pl.ds(0, NROW)], xcol, sem).wait()

    def fire(j, _):
        f = j // QROWS
        pltpu.async_copy(
            emb_hbm.at[xcol.at[j]], valv.at[pl.ds(j * 128, 128)], fsem.at[f]
        )
        return 0

    lax.fori_loop(0, NROW, fire, 0)

    zero = jnp.zeros((L,), jnp.float32)
    for cth in range(RPW // L):
        outv[pl.ds(cth * L, L)] = zero

    def acc(f, _):
        pltpu.make_async_copy(
            emb_hbm.at[pl.ds(0, RPW)], valv.at[pl.ds(f * RPW, RPW)], fsem.at[f]
        ).wait()
        for cth in range(RPW // L):
            outv[pl.ds(cth * L, L)] = (
                outv[pl.ds(cth * L, L)] + valv[pl.ds(f * RPW + cth * L, L)]
            )
        return 0

    lax.fori_loop(0, F, acc, 0)
    pltpu.sync_copy(outv, out_hbm.at[pl.ds(wid * RPW, RPW)])


def kernel(X, emb):
    offs = jnp.arange(F, dtype=jnp.int32)[:, None] * VP
    xt = (X.T.astype(jnp.int32) + offs).reshape(F * XROWS, 128)
    embp = jnp.pad(emb[:, :, 0], ((0, 0), (0, VP - V))).reshape(-1)
    mesh = plsc.VectorSubcoreMesh(
        core_axis_name="c", subcore_axis_name="s", num_cores=NC, num_subcores=NS
    )
    out = pl.kernel(
        _body,
        out_type=jax.ShapeDtypeStruct((B,), jnp.float32),
        mesh=mesh,
        scratch_types=[
            pltpu.VMEM((NROW, 128), jnp.int32),
            pltpu.VMEM((WORDS,), jnp.float32),
            pltpu.VMEM((RPW,), jnp.float32),
            pltpu.SemaphoreType.DMA,
            pltpu.SemaphoreType.DMA((F,)),
        ],
        compiler_params=pltpu.CompilerParams(needs_layout_passes=False),
    )(xt, embp)
    return out.reshape(B, 1)


# speedup vs baseline: 2.2775x; 1.0710x over previous
import jax
import jax.numpy as jnp
from jax import lax
from jax.experimental import pallas as pl
from jax.experimental.pallas import tpu as pltpu
from jax.experimental.pallas import tpu_sc as plsc

B = 16384
F = 26
V = 100000
VP = 100000

NC = 2
NS = 16
NW = NC * NS
RPW = B // NW
WORDS = RPW * F
QROWS = RPW // 128
NROW = F * QROWS
L = 16
XROWS = B // 128


def _body(xt_hbm, emb_hbm, out_hbm, xcol, valv, outv, sem, fsem):
    c = lax.axis_index("c")
    s = lax.axis_index("s")
    wid = s * NC + c

    def xfire(f, _):
        pltpu.async_copy(
            xt_hbm.at[pl.ds(f * XROWS + wid * QROWS, QROWS)],
            xcol.at[pl.ds(f * QROWS, QROWS)],
            sem,
        )
        return 0

    lax.fori_loop(0, F, xfire, 0)
    pltpu.make_async_copy(xt_hbm.at[pl.ds(0, NROW)], xcol, sem).wait()

    def fire(j, _):
        f = j // QROWS
        pltpu.async_copy(
            emb_hbm.at[xcol.at[j]], valv.at[pl.ds(j * 128, 128)], fsem.at[f]
        )
        return 0

    lax.fori_loop(0, NROW, fire, 0)

    zero = jnp.zeros((L,), jnp.float32)
    for cth in range(RPW // L):
        outv[pl.ds(cth * L, L)] = zero

    def acc(f, _):
        pltpu.make_async_copy(
            emb_hbm.at[pl.ds(0, RPW)], valv.at[pl.ds(f * RPW, RPW)], fsem.at[f]
        ).wait()
        for cth in range(RPW // L):
            outv[pl.ds(cth * L, L)] = (
                outv[pl.ds(cth * L, L)] + valv[pl.ds(f * RPW + cth * L, L)]
            )
        return 0

    lax.fori_loop(0, F, acc, 0)
    pltpu.sync_copy(outv, out_hbm.at[pl.ds(wid * RPW, RPW)])


def kernel(X, emb):
    offs = jnp.arange(F, dtype=jnp.int32)[:, None] * VP
    xt = (X.T.astype(jnp.int32) + offs).reshape(F * XROWS, 128)
    embp = lax.optimization_barrier(emb[:, :, 0]).reshape(-1)
    mesh = plsc.VectorSubcoreMesh(
        core_axis_name="c", subcore_axis_name="s", num_cores=NC, num_subcores=NS
    )
    out = pl.kernel(
        _body,
        out_type=jax.ShapeDtypeStruct((B,), jnp.float32),
        mesh=mesh,
        scratch_types=[
            pltpu.VMEM((NROW, 128), jnp.int32),
            pltpu.VMEM((WORDS,), jnp.float32),
            pltpu.VMEM((RPW,), jnp.float32),
            pltpu.SemaphoreType.DMA,
            pltpu.SemaphoreType.DMA((F,)),
        ],
        compiler_params=pltpu.CompilerParams(needs_layout_passes=False),
    )(xt, embp)
    return out.reshape(B, 1)
